# baseline (speedup 1.0000x reference)
import functools

import jax
import jax.numpy as jnp
from jax import lax
from jax.experimental import pallas as pl
from jax.experimental.pallas import tpu as pltpu
from jax.experimental.pallas import tpu_sc as plsc

L = 16384
DIM = 64
NC = 2
NS = 16
NW = NC * NS
PER_TILE = L // NW
CHUNK = 128
NCHUNK = PER_TILE // CHUNK


def _sc_partial_sums(token_ids, table):
    mesh = plsc.VectorSubcoreMesh(core_axis_name="c", subcore_axis_name="s")

    @functools.partial(
        pl.kernel,
        mesh=mesh,
        compiler_params=pltpu.CompilerParams(use_tc_tiling_on_sc=False),
        out_type=jax.ShapeDtypeStruct((NC, DIM), jnp.float32),
        scratch_types=[
            pltpu.VMEM((NCHUNK, CHUNK), jnp.int32),
            pltpu.VMEM((CHUNK, DIM), jnp.float32),
            pltpu.VMEM((CHUNK,), jnp.int32),
            pltpu.VMEM((DIM,), jnp.float32),
            pltpu.VMEM_SHARED((1, DIM), jnp.float32),
            pltpu.SemaphoreType.DMA,
        ],
    )
    def k(tok_hbm, table_hbm, out_hbm, idx_v, rows_v, zidx_v, zrow_v,
          acc_sh, sem):
        c = lax.axis_index("c")
        s = lax.axis_index("s")
        wid = s * NC + c
        base = wid * PER_TILE

        for j in range(NCHUNK):
            pltpu.sync_copy(tok_hbm.at[pl.ds(base + j * CHUNK, CHUNK)],
                            idx_v.at[j])

        z16 = jnp.zeros((16,), jnp.int32)
        for i in range(CHUNK // 16):
            zidx_v[pl.ds(i * 16, 16)] = z16
        zf16 = jnp.zeros((16,), jnp.float32)
        for i in range(DIM // 16):
            zrow_v[pl.ds(i * 16, 16)] = zf16

        @pl.when(s == 0)
        def _init():
            pltpu.sync_copy(zrow_v, acc_sh.at[0])

        plsc.subcore_barrier()

        for j in range(NCHUNK):
            pltpu.async_copy(table_hbm.at[idx_v.at[j]], rows_v, sem).wait()
            pltpu.sync_copy(rows_v, acc_sh.at[zidx_v], add=True)

        plsc.subcore_barrier()

        @pl.when(s == 0)
        def _emit():
            pltpu.sync_copy(acc_sh.at[0], out_hbm.at[c])

    return k(token_ids, table)


def _finish_body(p_ref, w_ref, o_ref):
    pooled = jnp.sum(p_ref[...], axis=0, keepdims=True) * (1.0 / L)
    o_ref[...] = lax.dot_general(
        pooled, w_ref[...], (((1,), (1,)), ((), ())),
        preferred_element_type=jnp.float32)


def kernel(token_ids, table, W):
    partials = _sc_partial_sums(token_ids, table)
    return pl.pallas_call(
        _finish_body,
        out_shape=jax.ShapeDtypeStruct((1, DIM), jnp.float32),
    )(partials, W)


# speedup vs baseline: 1.0034x; 1.0034x over previous
import functools

import jax
import jax.numpy as jnp
from jax import lax
from jax.experimental import pallas as pl
from jax.experimental.pallas import tpu as pltpu
from jax.experimental.pallas import tpu_sc as plsc

L = 16384
DIM = 64
NC = 2
NS = 16
NW = NC * NS
PER_TILE = L // NW
CHUNK = 128
NCHUNK = PER_TILE // CHUNK


def _sc_partial_sums(token_ids, table):
    mesh = plsc.VectorSubcoreMesh(core_axis_name="c", subcore_axis_name="s")

    @functools.partial(
        pl.kernel,
        mesh=mesh,
        compiler_params=pltpu.CompilerParams(use_tc_tiling_on_sc=False),
        out_type=jax.ShapeDtypeStruct((NC, DIM), jnp.float32),
        scratch_types=[
            pltpu.VMEM((PER_TILE,), jnp.int32),
            pltpu.VMEM((NCHUNK, CHUNK, DIM), jnp.float32),
            pltpu.VMEM((CHUNK,), jnp.int32),
            pltpu.VMEM((DIM,), jnp.float32),
            pltpu.VMEM_SHARED((1, DIM), jnp.float32),
            pltpu.SemaphoreType.DMA((NCHUNK,)),
            pltpu.SemaphoreType.DMA,
        ],
    )
    def k(tok_hbm, table_hbm, out_hbm, idx_v, rows_v, zidx_v, zrow_v,
          acc_sh, gsem, ssem):
        c = lax.axis_index("c")
        s = lax.axis_index("s")
        wid = s * NC + c
        base = wid * PER_TILE

        pltpu.sync_copy(tok_hbm.at[pl.ds(base, PER_TILE)], idx_v)

        gathers = [
            pltpu.async_copy(table_hbm.at[idx_v.at[pl.ds(j * CHUNK, CHUNK)]],
                             rows_v.at[j], gsem.at[j])
            for j in range(NCHUNK)
        ]

        z16 = jnp.zeros((16,), jnp.int32)
        for i in range(CHUNK // 16):
            zidx_v[pl.ds(i * 16, 16)] = z16
        zf16 = jnp.zeros((16,), jnp.float32)
        for i in range(DIM // 16):
            zrow_v[pl.ds(i * 16, 16)] = zf16

        @pl.when(s == 0)
        def _init():
            pltpu.sync_copy(zrow_v, acc_sh.at[0])

        plsc.subcore_barrier()

        scatters = []
        for j in range(NCHUNK):
            gathers[j].wait()
            scatters.append(
                pltpu.async_copy(rows_v.at[j], acc_sh.at[zidx_v], ssem,
                                 add=True))
        for sc in scatters:
            sc.wait()

        plsc.subcore_barrier()

        @pl.when(s == 0)
        def _emit():
            pltpu.sync_copy(acc_sh.at[0], out_hbm.at[c])

    return k(token_ids, table)


def _finish_body(p_ref, w_ref, o_ref):
    pooled = jnp.sum(p_ref[...], axis=0, keepdims=True) * (1.0 / L)
    o_ref[...] = lax.dot_general(
        pooled, w_ref[...], (((1,), (1,)), ((), ())),
        preferred_element_type=jnp.float32)


def kernel(token_ids, table, W):
    partials = _sc_partial_sums(token_ids, table)
    return pl.pallas_call(
        _finish_body,
        out_shape=jax.ShapeDtypeStruct((1, DIM), jnp.float32),
    )(partials, W)


# speedup vs baseline: 1.7369x; 1.7310x over previous
import functools

import jax
import jax.numpy as jnp
from jax import lax
from jax.experimental import pallas as pl
from jax.experimental.pallas import tpu as pltpu
from jax.experimental.pallas import tpu_sc as plsc

L = 16384
DIM = 64
NC = 2
NS = 16
NW = NC * NS
PER_TILE = L // NW
CHUNK = 128
NCHUNK = PER_TILE // CHUNK
NBUF = 2


def _sc_partial_sums(token_ids, table):
    mesh = plsc.VectorSubcoreMesh(core_axis_name="c", subcore_axis_name="s")

    @functools.partial(
        pl.kernel,
        mesh=mesh,
        out_type=jax.ShapeDtypeStruct((2 * 128,), jnp.float32),
        scratch_types=[
            pltpu.VMEM((PER_TILE,), jnp.int32),
            pltpu.VMEM((NBUF, CHUNK, DIM), jnp.float32),
            pltpu.VMEM((128,), jnp.float32),
            pltpu.VMEM((NS * DIM,), jnp.float32),
            pltpu.VMEM_SHARED((NS * DIM,), jnp.float32),
            pltpu.SemaphoreType.DMA((NBUF,)),
        ],
    )
    def k(tok_hbm, tab_hbm, out_hbm, idx_v, rows_v, acc_v, sum_v,
          acc_sh, gsem):
        c = lax.axis_index("c")
        s = lax.axis_index("s")
        wid = s * NC + c
        base = wid * PER_TILE

        pltpu.sync_copy(tok_hbm.at[pl.ds(base, PER_TILE)], idx_v)

        def fire_chunk(j):
            b = j % NBUF

            def grp(g, _):
                toks = idx_v[pl.ds(j * CHUNK + g * 16, 16)]
                for lane in range(16):
                    pltpu.async_copy(
                        tab_hbm.at[pl.ds(toks[lane], 1)],
                        rows_v.at[b].at[pl.ds(g * 16 + lane, 1)],
                        gsem.at[b])
                return 0

            lax.fori_loop(0, CHUNK // 16, grp, 0)

        def drain_chunk(j):
            b = j % NBUF

            def one(t, _):
                pltpu.make_async_copy(tab_hbm.at[pl.ds(0, 1)],
                                      rows_v.at[b].at[pl.ds(0, 1)],
                                      gsem.at[b]).wait()
                return 0

            lax.fori_loop(0, CHUNK, one, 0)

        def accum_chunk(j, carry):
            b = j % NBUF

            def body(t, cr):
                return (
                    cr[0] + rows_v[b, t, pl.ds(0, 16)],
                    cr[1] + rows_v[b, t, pl.ds(16, 16)],
                    cr[2] + rows_v[b, t, pl.ds(32, 16)],
                    cr[3] + rows_v[b, t, pl.ds(48, 16)],
                )

            return lax.fori_loop(0, CHUNK, body, carry)

        for j in range(min(NBUF, NCHUNK)):
            fire_chunk(j)
        zero16 = jnp.zeros((16,), jnp.float32)
        carry = (zero16, zero16, zero16, zero16)
        for j in range(NCHUNK):
            drain_chunk(j)
            carry = accum_chunk(j, carry)
            if j + NBUF < NCHUNK:
                fire_chunk(j + NBUF)

        for g in range(4):
            acc_v[pl.ds(g * 16, 16)] = carry[g]
        pltpu.sync_copy(acc_v.at[pl.ds(0, DIM)],
                        acc_sh.at[pl.ds(s * DIM, DIM)])

        plsc.subcore_barrier()

        @pl.when(s == 0)
        def _emit():
            pltpu.sync_copy(acc_sh, sum_v)
            tot = [jnp.zeros((16,), jnp.float32) for _ in range(4)]
            for ss in range(NS):
                for g in range(4):
                    tot[g] = tot[g] + sum_v[pl.ds(ss * DIM + g * 16, 16)]
            for g in range(4):
                acc_v[pl.ds(g * 16, 16)] = tot[g]
            z16 = jnp.zeros((16,), jnp.float32)
            for g in range(4, 8):
                acc_v[pl.ds(g * 16, 16)] = z16
            pltpu.sync_copy(acc_v, out_hbm.at[pl.ds(c * 128, 128)])

    return k(token_ids, table)


def _finish_body(p_ref, w_ref, o_ref):
    x = p_ref[...]
    pooled = ((x[0:DIM] + x[128:128 + DIM]) * (1.0 / L)).reshape(1, DIM)
    o_ref[...] = lax.dot_general(
        pooled, w_ref[...], (((1,), (1,)), ((), ())),
        preferred_element_type=jnp.float32)


def kernel(token_ids, table, W):
    partials = _sc_partial_sums(token_ids, table)
    return pl.pallas_call(
        _finish_body,
        out_shape=jax.ShapeDtypeStruct((1, DIM), jnp.float32),
    )(partials, W)


# speedup vs baseline: 3.7372x; 2.1517x over previous
import functools

import jax
import jax.numpy as jnp
from jax import lax
from jax.experimental import pallas as pl
from jax.experimental.pallas import tpu as pltpu
from jax.experimental.pallas import tpu_sc as plsc

L = 16384
DIM = 64
VOCAB = 1000000
HIST = 1 << 20
NC = 2
NS = 16
NW = NC * NS
PER_TILE = L // NW
SCHUNK = 128
NSCHUNK = PER_TILE // SCHUNK
ZBUF = 8192
ZREP = HIST // NS // ZBUF

BLK = 8192
NBLK = -(-VOCAB // BLK)


def _sc_histograms(token_ids):
    mesh = plsc.VectorSubcoreMesh(core_axis_name="c", subcore_axis_name="s")

    @functools.partial(
        pl.kernel,
        mesh=mesh,
        out_type=(jax.ShapeDtypeStruct((HIST,), jnp.float32),
                  jax.ShapeDtypeStruct((HIST,), jnp.float32)),
        scratch_types=[
            pltpu.VMEM((NSCHUNK, SCHUNK), jnp.int32),
            pltpu.VMEM((SCHUNK,), jnp.float32),
            pltpu.VMEM((ZBUF,), jnp.float32),
            pltpu.VMEM_SHARED((HIST,), jnp.float32),
            pltpu.SemaphoreType.DMA,
        ],
    )
    def k(tok_hbm, out0_hbm, out1_hbm, idx_v, ones_v, zbuf_v, hist_sh, zsem):
        c = lax.axis_index("c")
        s = lax.axis_index("s")
        wid = s * NC + c
        base = wid * PER_TILE

        for q in range(NSCHUNK):
            pltpu.sync_copy(tok_hbm.at[pl.ds(base + q * SCHUNK, SCHUNK)],
                            idx_v.at[q])

        one16 = jnp.full((16,), 1.0, jnp.float32)
        for i in range(SCHUNK // 16):
            ones_v[pl.ds(i * 16, 16)] = one16

        z16 = jnp.zeros((16,), jnp.float32)

        def zfill(i, _):
            zbuf_v[pl.ds(i * 16, 16)] = z16
            return 0

        lax.fori_loop(0, ZBUF // 16, zfill, 0)

        zdescs = [
            pltpu.async_copy(
                zbuf_v,
                hist_sh.at[pl.ds((s * ZREP + r) * ZBUF, ZBUF)],
                zsem)
            for r in range(ZREP)
        ]
        for d in zdescs:
            d.wait()

        plsc.subcore_barrier()

        for q in range(NSCHUNK):
            pltpu.sync_copy(ones_v, hist_sh.at[idx_v.at[q]], add=True)

        plsc.subcore_barrier()

        @pl.when(jnp.logical_and(s == 0, c == 0))
        def _emit0():
            pltpu.sync_copy(hist_sh, out0_hbm)

        @pl.when(jnp.logical_and(s == 0, c == 1))
        def _emit1():
            pltpu.sync_copy(hist_sh, out1_hbm)

    return k(token_ids)


def _scan_body(tab_ref, c0_ref, c1_ref, w_ref, o_ref, acc_ref):
    i = pl.program_id(0)

    @pl.when(i == 0)
    def _init():
        acc_ref[...] = jnp.zeros_like(acc_ref)

    cnt = c0_ref[...] + c1_ref[...]
    tab = tab_ref[...]

    @pl.when(i == NBLK - 1)
    def _mask_tail():
        col = i * BLK + lax.broadcasted_iota(jnp.int32, (DIM, BLK), 1)
        tab_ref[...] = jnp.where(col < VOCAB, tab, 0.0)

    contrib = lax.dot_general(
        tab_ref[...], cnt, (((1,), (0,)), ((), ())),
        preferred_element_type=jnp.float32)
    acc_ref[...] += contrib.reshape(1, DIM)

    @pl.when(i == NBLK - 1)
    def _fin():
        pooled = acc_ref[...] * (1.0 / L)
        o_ref[...] = lax.dot_general(
            pooled, w_ref[...], (((1,), (1,)), ((), ())),
            preferred_element_type=jnp.float32)


def kernel(token_ids, table, W):
    c0, c1 = _sc_histograms(token_ids)
    tab_t = table.T
    return pl.pallas_call(
        _scan_body,
        grid=(NBLK,),
        in_specs=[
            pl.BlockSpec((DIM, BLK), lambda i: (0, i)),
            pl.BlockSpec((BLK,), lambda i: (i,)),
            pl.BlockSpec((BLK,), lambda i: (i,)),
            pl.BlockSpec((DIM, DIM), lambda i: (0, 0)),
        ],
        out_specs=pl.BlockSpec((1, DIM), lambda i: (0, 0)),
        out_shape=jax.ShapeDtypeStruct((1, DIM), jnp.float32),
        scratch_shapes=[pltpu.VMEM((1, DIM), jnp.float32)],
    )(tab_t, c0, c1, W)
